# Initial kernel scaffold; baseline (speedup 1.0000x reference)
import jax
import jax.numpy as jnp
from jax.experimental import pallas as pl


def kernel(x, edge_index, edge_attr, u, batch, We1, be1, We2, be2, We3, be3, We4, be4, Wn1, bn1, Wn2, bn2, Wn3, bn3, Wn4, bn4):
    raise NotImplementedError("write your pallas kernel here")


# speedup vs baseline: 1.7286x; 1.7286x over previous
import functools
import jax
import jax.numpy as jnp
from jax import lax
from jax.experimental import pallas as pl
from jax.experimental.pallas import tpu as pltpu, tpu_sc as plsc

N = 10000
E = 320000
F = 128
NC = 2
NS = 16
NW = NC * NS
EPW = E // NW
GB = 80
NCHUNK = EPW // GB
NPAD = 10240
NPT = NPAD // NS

_mesh = plsc.VectorSubcoreMesh(core_axis_name="c", subcore_axis_name="s")


@functools.partial(
    pl.kernel,
    out_type=(
        jax.ShapeDtypeStruct((E, F), jnp.float32),
        jax.ShapeDtypeStruct((E, F), jnp.float32),
    ),
    mesh=_mesh,
    scratch_types=[
        pltpu.VMEM((GB,), jnp.int32),
        pltpu.VMEM((GB,), jnp.int32),
        pltpu.VMEM((GB, F), jnp.float32),
        pltpu.VMEM((GB, F), jnp.float32),
        pltpu.SemaphoreType.DMA,
        pltpu.SemaphoreType.DMA,
    ],
)
def _sc_gather(tab_r, tab_c, row, col, gr, gc, idx_r, idx_c, buf_r, buf_c,
               sem_r, sem_c):
    cid = lax.axis_index("c")
    sid = lax.axis_index("s")
    wid = cid * NS + sid
    base_w = wid * EPW

    def body(i, _):
        base = pl.multiple_of(base_w + i * GB, GB)
        pltpu.sync_copy(row.at[pl.ds(base, GB)], idx_r)
        pltpu.sync_copy(col.at[pl.ds(base, GB)], idx_c)
        cp_r = pltpu.async_copy(tab_r.at[idx_r], buf_r, sem_r)
        cp_c = pltpu.async_copy(tab_c.at[idx_c], buf_c, sem_c)
        cp_r.wait()
        cp_c.wait()
        pltpu.sync_copy(buf_r, gr.at[pl.ds(base, GB)])
        pltpu.sync_copy(buf_c, gc.at[pl.ds(base, GB)])
        return 0

    lax.fori_loop(0, NCHUNK, body, 0)


@functools.partial(
    pl.kernel,
    out_type=jax.ShapeDtypeStruct((NC, NPAD, F), jnp.float32),
    mesh=_mesh,
    scratch_types=[
        pltpu.VMEM((GB,), jnp.int32),
        pltpu.VMEM((GB, F), jnp.float32),
        pltpu.VMEM_SHARED((NPAD, F), jnp.float32),
        pltpu.SemaphoreType.DMA,
    ],
)
def _sc_scatter(ne, col, zsum, psum, idx_v, buf, acc, sem):
    cid = lax.axis_index("c")
    sid = lax.axis_index("s")
    nbase = pl.multiple_of(sid * NPT, NPT)

    def zinit(j, _):
        b = pl.multiple_of(nbase + j * GB, GB)
        pltpu.sync_copy(zsum.at[pl.ds(b, GB)], buf)
        pltpu.sync_copy(buf, acc.at[pl.ds(b, GB)])
        return 0

    lax.fori_loop(0, NPT // GB, zinit, 0)
    plsc.subcore_barrier()

    base_w = (cid * NS + sid) * EPW

    def body(i, _):
        base = pl.multiple_of(base_w + i * GB, GB)
        pltpu.sync_copy(col.at[pl.ds(base, GB)], idx_v)
        cp = pltpu.async_copy(ne.at[pl.ds(base, GB)], buf, sem)
        cp.wait()
        pltpu.sync_copy(buf, acc.at[idx_v], add=True)
        return 0

    lax.fori_loop(0, NCHUNK, body, 0)
    plsc.subcore_barrier()

    def wb(j, _):
        b = pl.multiple_of(nbase + j * GB, GB)
        pltpu.sync_copy(acc.at[pl.ds(b, GB)], buf)
        pltpu.sync_copy(buf, psum.at[cid, pl.ds(b, GB)])
        return 0

    lax.fori_loop(0, NPT // GB, wb, 0)


@functools.partial(
    pl.kernel,
    out_type=jax.ShapeDtypeStruct((NC, NPAD, F), jnp.float32),
    mesh=_mesh,
    scratch_types=[
        pltpu.VMEM((GB,), jnp.int32),
        pltpu.VMEM((GB, F), jnp.float32),
        pltpu.VMEM((GB, F), jnp.float32),
        pltpu.VMEM_SHARED((NPAD, F), jnp.float32),
        pltpu.SemaphoreType.DMA,
    ],
)
def _sc_count(col, zsum, ones, pcnt, idx_v, buf, ones_v, acc, sem):
    cid = lax.axis_index("c")
    sid = lax.axis_index("s")
    nbase = pl.multiple_of(sid * NPT, NPT)

    def zinit(j, _):
        b = pl.multiple_of(nbase + j * GB, GB)
        pltpu.sync_copy(zsum.at[pl.ds(b, GB)], buf)
        pltpu.sync_copy(buf, acc.at[pl.ds(b, GB)])
        return 0

    lax.fori_loop(0, NPT // GB, zinit, 0)
    pltpu.sync_copy(ones, ones_v)
    plsc.subcore_barrier()

    base_w = (cid * NS + sid) * EPW

    def body(i, _):
        base = pl.multiple_of(base_w + i * GB, GB)
        pltpu.sync_copy(col.at[pl.ds(base, GB)], idx_v)
        pltpu.sync_copy(ones_v, acc.at[idx_v], add=True)
        return 0

    lax.fori_loop(0, NCHUNK, body, 0)
    plsc.subcore_barrier()

    def wb(j, _):
        b = pl.multiple_of(nbase + j * GB, GB)
        pltpu.sync_copy(acc.at[pl.ds(b, GB)], buf)
        pltpu.sync_copy(buf, pcnt.at[cid, pl.ds(b, GB)])
        return 0

    lax.fori_loop(0, NPT // GB, wb, 0)


_PREC = lax.Precision.HIGHEST


def _dot(a, b):
    return jnp.dot(a, b, preferred_element_type=jnp.float32, precision=_PREC)


def _proj_body(x_ref, wa_ref, wb_ref, xr_ref, xc_ref):
    x = x_ref[...]
    xr_ref[...] = _dot(x, wa_ref[...])
    xc_ref[...] = _dot(x, wb_ref[...])


def _edge_body(gr_ref, gc_ref, ea_ref, w1_ref, b1_ref, w2_ref, b2_ref,
               w3_ref, b3_ref, w4_ref, b4_ref, ne_ref, eo_ref):
    ea = ea_ref[...]
    h = jnp.maximum(gr_ref[...] + gc_ref[...] + _dot(ea, w1_ref[...])
                    + b1_ref[...], 0.0)
    h = jnp.maximum(_dot(h, w2_ref[...]) + b2_ref[...], 0.0)
    h = jnp.maximum(_dot(h, w3_ref[...]) + b3_ref[...], 0.0)
    ne = _dot(h, w4_ref[...]) + b4_ref[...]
    ne_ref[...] = ne
    eo_ref[...] = ea + ne


def _node_body(x_ref, p0_ref, p1_ref, c0_ref, c1_ref, wa_ref, wb_ref, b1_ref,
               w2_ref, b2_ref, w3_ref, b3_ref, w4_ref, b4_ref, no_ref):
    x = x_ref[...]
    cnt = jnp.maximum(c0_ref[...][:, :1] + c1_ref[...][:, :1], 1.0)
    agg = (p0_ref[...] + p1_ref[...]) / cnt
    h = jnp.maximum(_dot(x, wa_ref[...]) + _dot(agg, wb_ref[...])
                    + b1_ref[...], 0.0)
    h = jnp.maximum(_dot(h, w2_ref[...]) + b2_ref[...], 0.0)
    h = jnp.maximum(_dot(h, w3_ref[...]) + b3_ref[...], 0.0)
    no_ref[...] = x + _dot(h, w4_ref[...]) + b4_ref[...]


def _wspec(shape):
    return pl.BlockSpec(shape, lambda i: (0,) * len(shape))


def kernel(x, edge_index, edge_attr, u, batch, We1, be1, We2, be2, We3, be3,
           We4, be4, Wn1, bn1, Wn2, bn2, Wn3, bn3, Wn4, bn4):
    row = edge_index[0]
    col = edge_index[1]

    BN = 1000
    xr, xc = pl.pallas_call(
        _proj_body,
        grid=(N // BN,),
        in_specs=[
            pl.BlockSpec((BN, F), lambda i: (i, 0)),
            _wspec((F, F)),
            _wspec((F, F)),
        ],
        out_specs=[
            pl.BlockSpec((BN, F), lambda i: (i, 0)),
            pl.BlockSpec((BN, F), lambda i: (i, 0)),
        ],
        out_shape=[
            jax.ShapeDtypeStruct((N, F), jnp.float32),
            jax.ShapeDtypeStruct((N, F), jnp.float32),
        ],
    )(x, We1[:F], We1[F:2 * F])

    gr, gc = _sc_gather(xr, xc, row, col)

    zsum = jnp.zeros((NPAD, F), jnp.float32)
    ones = jnp.ones((GB, F), jnp.float32)
    pcnt = _sc_count(col, zsum, ones)

    BE = 2000
    eb = pl.BlockSpec((BE, F), lambda i: (i, 0))
    bias = _wspec((1, F))
    ne, eout = pl.pallas_call(
        _edge_body,
        grid=(E // BE,),
        in_specs=[eb, eb, eb, _wspec((F, F)), bias, _wspec((F, F)), bias,
                  _wspec((F, F)), bias, _wspec((F, F)), bias],
        out_specs=[eb, eb],
        out_shape=[
            jax.ShapeDtypeStruct((E, F), jnp.float32),
            jax.ShapeDtypeStruct((E, F), jnp.float32),
        ],
    )(gr, gc, edge_attr, We1[2 * F:], be1[None], We2, be2[None], We3,
      be3[None], We4, be4[None])

    psum = _sc_scatter(ne, col, zsum)

    nb = pl.BlockSpec((BN, F), lambda i: (i, 0))
    nout = pl.pallas_call(
        _node_body,
        grid=(N // BN,),
        in_specs=[nb, nb, nb, nb, nb, _wspec((F, F)), _wspec((F, F)), bias,
                  _wspec((F, F)), bias, _wspec((F, F)), bias, _wspec((F, F)),
                  bias],
        out_specs=nb,
        out_shape=jax.ShapeDtypeStruct((N, F), jnp.float32),
    )(x, psum[0, :N], psum[1, :N], pcnt[0, :N], pcnt[1, :N], Wn1[:F],
      Wn1[F:], bn1[None], Wn2, bn2[None], Wn3, bn3[None], Wn4, bn4[None])

    return nout, eout


# speedup vs baseline: 1.8457x; 1.0677x over previous
import functools
import jax
import jax.numpy as jnp
from jax import lax
from jax.experimental import pallas as pl
from jax.experimental.pallas import tpu as pltpu, tpu_sc as plsc

N = 10000
E = 320000
F = 128
NC = 2
NS = 16
NW = NC * NS
EPW = E // NW
GB = 80
NCHUNK = EPW // GB
NPAD = 10240
NPT = NPAD // NS

_mesh = plsc.VectorSubcoreMesh(core_axis_name="c", subcore_axis_name="s")


@functools.partial(
    pl.kernel,
    out_type=(
        jax.ShapeDtypeStruct((E, F), jnp.float32),
        jax.ShapeDtypeStruct((E, F), jnp.float32),
    ),
    mesh=_mesh,
    scratch_types=[
        pltpu.VMEM((GB,), jnp.int32),
        pltpu.VMEM((GB,), jnp.int32),
        pltpu.VMEM((GB,), jnp.int32),
        pltpu.VMEM((GB,), jnp.int32),
        pltpu.VMEM((GB, F), jnp.float32),
        pltpu.VMEM((GB, F), jnp.float32),
        pltpu.VMEM((GB, F), jnp.float32),
        pltpu.VMEM((GB, F), jnp.float32),
        pltpu.SemaphoreType.DMA,
        pltpu.SemaphoreType.DMA,
        pltpu.SemaphoreType.DMA,
        pltpu.SemaphoreType.DMA,
    ],
)
def _sc_gather(tab_r, tab_c, row, col, gr, gc, idx_r0, idx_c0, idx_r1,
               idx_c1, buf_r0, buf_c0, buf_r1, buf_c1, sem_r0, sem_c0,
               sem_r1, sem_c1):
    cid = lax.axis_index("c")
    sid = lax.axis_index("s")
    base_w = (cid * NS + sid) * EPW

    def chunk(base, idx_r, idx_c, buf_r, buf_c, sem_r, sem_c):
        pltpu.sync_copy(row.at[pl.ds(base, GB)], idx_r)
        pltpu.sync_copy(col.at[pl.ds(base, GB)], idx_c)
        cp_r = pltpu.async_copy(tab_r.at[idx_r], buf_r, sem_r)
        cp_c = pltpu.async_copy(tab_c.at[idx_c], buf_c, sem_c)
        return cp_r, cp_c

    def drain(base, buf_r, buf_c, cps):
        cps[0].wait()
        cps[1].wait()
        pltpu.sync_copy(buf_r, gr.at[pl.ds(base, GB)])
        pltpu.sync_copy(buf_c, gc.at[pl.ds(base, GB)])

    def body(i, _):
        base_a = pl.multiple_of(base_w + (2 * i) * GB, GB)
        base_b = pl.multiple_of(base_w + (2 * i + 1) * GB, GB)
        cps_a = chunk(base_a, idx_r0, idx_c0, buf_r0, buf_c0, sem_r0, sem_c0)
        cps_b = chunk(base_b, idx_r1, idx_c1, buf_r1, buf_c1, sem_r1, sem_c1)
        drain(base_a, buf_r0, buf_c0, cps_a)
        drain(base_b, buf_r1, buf_c1, cps_b)
        return 0

    lax.fori_loop(0, NCHUNK // 2, body, 0)
    base_l = pl.multiple_of(base_w + (NCHUNK - 1) * GB, GB)
    cps_l = chunk(base_l, idx_r0, idx_c0, buf_r0, buf_c0, sem_r0, sem_c0)
    drain(base_l, buf_r0, buf_c0, cps_l)


@functools.partial(
    pl.kernel,
    out_type=jax.ShapeDtypeStruct((NC, NPAD, F), jnp.float32),
    mesh=_mesh,
    scratch_types=[
        pltpu.VMEM((GB,), jnp.int32),
        pltpu.VMEM((GB, F), jnp.float32),
        pltpu.VMEM_SHARED((NPAD, F), jnp.float32),
        pltpu.SemaphoreType.DMA,
    ],
)
def _sc_scatter(ne, col, zsum, psum, idx_v, buf, acc, sem):
    cid = lax.axis_index("c")
    sid = lax.axis_index("s")
    nbase = pl.multiple_of(sid * NPT, NPT)

    def zinit(j, _):
        b = pl.multiple_of(nbase + j * GB, GB)
        pltpu.sync_copy(zsum.at[pl.ds(b, GB)], buf)
        pltpu.sync_copy(buf, acc.at[pl.ds(b, GB)])
        return 0

    lax.fori_loop(0, NPT // GB, zinit, 0)
    plsc.subcore_barrier()

    base_w = (cid * NS + sid) * EPW

    def body(i, _):
        base = pl.multiple_of(base_w + i * GB, GB)
        pltpu.sync_copy(col.at[pl.ds(base, GB)], idx_v)
        cp = pltpu.async_copy(ne.at[pl.ds(base, GB)], buf, sem)
        cp.wait()
        pltpu.sync_copy(buf, acc.at[idx_v], add=True)
        return 0

    lax.fori_loop(0, NCHUNK, body, 0)
    plsc.subcore_barrier()

    def wb(j, _):
        b = pl.multiple_of(nbase + j * GB, GB)
        pltpu.sync_copy(acc.at[pl.ds(b, GB)], buf)
        pltpu.sync_copy(buf, psum.at[cid, pl.ds(b, GB)])
        return 0

    lax.fori_loop(0, NPT // GB, wb, 0)


@functools.partial(
    pl.kernel,
    out_type=jax.ShapeDtypeStruct((NC, NPAD, F), jnp.float32),
    mesh=_mesh,
    scratch_types=[
        pltpu.VMEM((GB,), jnp.int32),
        pltpu.VMEM((GB, F), jnp.float32),
        pltpu.VMEM((GB, F), jnp.float32),
        pltpu.VMEM_SHARED((NPAD, F), jnp.float32),
        pltpu.SemaphoreType.DMA,
    ],
)
def _sc_count(col, zsum, ones, pcnt, idx_v, buf, ones_v, acc, sem):
    cid = lax.axis_index("c")
    sid = lax.axis_index("s")
    nbase = pl.multiple_of(sid * NPT, NPT)

    def zinit(j, _):
        b = pl.multiple_of(nbase + j * GB, GB)
        pltpu.sync_copy(zsum.at[pl.ds(b, GB)], buf)
        pltpu.sync_copy(buf, acc.at[pl.ds(b, GB)])
        return 0

    lax.fori_loop(0, NPT // GB, zinit, 0)
    pltpu.sync_copy(ones, ones_v)
    plsc.subcore_barrier()

    base_w = (cid * NS + sid) * EPW

    def body(i, _):
        base = pl.multiple_of(base_w + i * GB, GB)
        pltpu.sync_copy(col.at[pl.ds(base, GB)], idx_v)
        pltpu.sync_copy(ones_v, acc.at[idx_v], add=True)
        return 0

    lax.fori_loop(0, NCHUNK, body, 0)
    plsc.subcore_barrier()

    def wb(j, _):
        b = pl.multiple_of(nbase + j * GB, GB)
        pltpu.sync_copy(acc.at[pl.ds(b, GB)], buf)
        pltpu.sync_copy(buf, pcnt.at[cid, pl.ds(b, GB)])
        return 0

    lax.fori_loop(0, NPT // GB, wb, 0)


_PREC = lax.Precision.HIGHEST


def _dot(a, b):
    return jnp.dot(a, b, preferred_element_type=jnp.float32, precision=_PREC)


def _proj_body(x_ref, wa_ref, wb_ref, xr_ref, xc_ref):
    x = x_ref[...]
    xr_ref[...] = _dot(x, wa_ref[...])
    xc_ref[...] = _dot(x, wb_ref[...])


def _edge_body(gr_ref, gc_ref, ea_ref, w1_ref, b1_ref, w2_ref, b2_ref,
               w3_ref, b3_ref, w4_ref, b4_ref, ne_ref, eo_ref):
    ea = ea_ref[...]
    h = jnp.maximum(gr_ref[...] + gc_ref[...] + _dot(ea, w1_ref[...])
                    + b1_ref[...], 0.0)
    h = jnp.maximum(_dot(h, w2_ref[...]) + b2_ref[...], 0.0)
    h = jnp.maximum(_dot(h, w3_ref[...]) + b3_ref[...], 0.0)
    ne = _dot(h, w4_ref[...]) + b4_ref[...]
    ne_ref[...] = ne
    eo_ref[...] = ea + ne


def _node_body(x_ref, p0_ref, p1_ref, c0_ref, c1_ref, wa_ref, wb_ref, b1_ref,
               w2_ref, b2_ref, w3_ref, b3_ref, w4_ref, b4_ref, no_ref):
    x = x_ref[...]
    cnt = jnp.maximum(c0_ref[...][:, :1] + c1_ref[...][:, :1], 1.0)
    agg = (p0_ref[...] + p1_ref[...]) / cnt
    h = jnp.maximum(_dot(x, wa_ref[...]) + _dot(agg, wb_ref[...])
                    + b1_ref[...], 0.0)
    h = jnp.maximum(_dot(h, w2_ref[...]) + b2_ref[...], 0.0)
    h = jnp.maximum(_dot(h, w3_ref[...]) + b3_ref[...], 0.0)
    no_ref[...] = x + _dot(h, w4_ref[...]) + b4_ref[...]


def _wspec(shape):
    return pl.BlockSpec(shape, lambda i: (0,) * len(shape))


def kernel(x, edge_index, edge_attr, u, batch, We1, be1, We2, be2, We3, be3,
           We4, be4, Wn1, bn1, Wn2, bn2, Wn3, bn3, Wn4, bn4):
    row = edge_index[0]
    col = edge_index[1]

    BN = 1000
    xr, xc = pl.pallas_call(
        _proj_body,
        grid=(N // BN,),
        in_specs=[
            pl.BlockSpec((BN, F), lambda i: (i, 0)),
            _wspec((F, F)),
            _wspec((F, F)),
        ],
        out_specs=[
            pl.BlockSpec((BN, F), lambda i: (i, 0)),
            pl.BlockSpec((BN, F), lambda i: (i, 0)),
        ],
        out_shape=[
            jax.ShapeDtypeStruct((N, F), jnp.float32),
            jax.ShapeDtypeStruct((N, F), jnp.float32),
        ],
    )(x, We1[:F], We1[F:2 * F])

    gr, gc = _sc_gather(xr, xc, row, col)

    zsum = jnp.zeros((NPAD, F), jnp.float32)
    ones = jnp.ones((GB, F), jnp.float32)
    pcnt = _sc_count(col, zsum, ones)

    BE = 2000
    eb = pl.BlockSpec((BE, F), lambda i: (i, 0))
    bias = _wspec((1, F))
    ne, eout = pl.pallas_call(
        _edge_body,
        grid=(E // BE,),
        in_specs=[eb, eb, eb, _wspec((F, F)), bias, _wspec((F, F)), bias,
                  _wspec((F, F)), bias, _wspec((F, F)), bias],
        out_specs=[eb, eb],
        out_shape=[
            jax.ShapeDtypeStruct((E, F), jnp.float32),
            jax.ShapeDtypeStruct((E, F), jnp.float32),
        ],
    )(gr, gc, edge_attr, We1[2 * F:], be1[None], We2, be2[None], We3,
      be3[None], We4, be4[None])

    psum = _sc_scatter(ne, col, zsum)

    nb = pl.BlockSpec((BN, F), lambda i: (i, 0))
    nout = pl.pallas_call(
        _node_body,
        grid=(N // BN,),
        in_specs=[nb, nb, nb, nb, nb, _wspec((F, F)), _wspec((F, F)), bias,
                  _wspec((F, F)), bias, _wspec((F, F)), bias, _wspec((F, F)),
                  bias],
        out_specs=nb,
        out_shape=jax.ShapeDtypeStruct((N, F), jnp.float32),
    )(x, psum[0, :N], psum[1, :N], pcnt[0, :N], pcnt[1, :N], Wn1[:F],
      Wn1[F:], bn1[None], Wn2, bn2[None], Wn3, bn3[None], Wn4, bn4[None])

    return nout, eout


# speedup vs baseline: 3.2987x; 1.7873x over previous
import functools
import jax
import jax.numpy as jnp
from jax import lax
from jax.experimental import pallas as pl
from jax.experimental.pallas import tpu as pltpu, tpu_sc as plsc

N = 10000
E = 320000
F = 128
NC = 2
NS = 16
NW = NC * NS
EPW = E // NW
GB = 80
NCHUNK = EPW // GB
NPAD = 10240
NPT = NPAD // NS

_mesh = plsc.VectorSubcoreMesh(core_axis_name="c", subcore_axis_name="s")


@functools.partial(
    pl.kernel,
    out_type=(
        jax.ShapeDtypeStruct((E, F), jnp.float32),
        jax.ShapeDtypeStruct((E, F), jnp.float32),
    ),
    mesh=_mesh,
    scratch_types=[
        pltpu.VMEM((GB,), jnp.int32),
        pltpu.VMEM((GB,), jnp.int32),
        pltpu.VMEM((GB,), jnp.int32),
        pltpu.VMEM((GB,), jnp.int32),
        pltpu.VMEM((GB, F), jnp.float32),
        pltpu.VMEM((GB, F), jnp.float32),
        pltpu.VMEM((GB, F), jnp.float32),
        pltpu.VMEM((GB, F), jnp.float32),
        pltpu.SemaphoreType.DMA,
        pltpu.SemaphoreType.DMA,
        pltpu.SemaphoreType.DMA,
        pltpu.SemaphoreType.DMA,
    ],
)
def _sc_gather(tab_r, tab_c, row, col, gr, gc, idx_r0, idx_c0, idx_r1,
               idx_c1, buf_r0, buf_c0, buf_r1, buf_c1, sem_r0, sem_c0,
               sem_r1, sem_c1):
    cid = lax.axis_index("c")
    sid = lax.axis_index("s")
    base_w = (cid * NS + sid) * EPW

    def chunk(base, idx_r, idx_c, buf_r, buf_c, sem_r, sem_c):
        pltpu.sync_copy(row.at[pl.ds(base, GB)], idx_r)
        pltpu.sync_copy(col.at[pl.ds(base, GB)], idx_c)
        cp_r = pltpu.async_copy(tab_r.at[idx_r], buf_r, sem_r)
        cp_c = pltpu.async_copy(tab_c.at[idx_c], buf_c, sem_c)
        return cp_r, cp_c

    def drain(base, buf_r, buf_c, cps):
        cps[0].wait()
        cps[1].wait()
        pltpu.sync_copy(buf_r, gr.at[pl.ds(base, GB)])
        pltpu.sync_copy(buf_c, gc.at[pl.ds(base, GB)])

    def body(i, _):
        base_a = pl.multiple_of(base_w + (2 * i) * GB, GB)
        base_b = pl.multiple_of(base_w + (2 * i + 1) * GB, GB)
        cps_a = chunk(base_a, idx_r0, idx_c0, buf_r0, buf_c0, sem_r0, sem_c0)
        cps_b = chunk(base_b, idx_r1, idx_c1, buf_r1, buf_c1, sem_r1, sem_c1)
        drain(base_a, buf_r0, buf_c0, cps_a)
        drain(base_b, buf_r1, buf_c1, cps_b)
        return 0

    lax.fori_loop(0, NCHUNK // 2, body, 0)
    base_l = pl.multiple_of(base_w + (NCHUNK - 1) * GB, GB)
    cps_l = chunk(base_l, idx_r0, idx_c0, buf_r0, buf_c0, sem_r0, sem_c0)
    drain(base_l, buf_r0, buf_c0, cps_l)


@functools.partial(
    pl.kernel,
    out_type=jax.ShapeDtypeStruct((NC, NPAD, F), jnp.float32),
    mesh=_mesh,
    scratch_types=[
        pltpu.VMEM((GB,), jnp.int32),
        pltpu.VMEM((GB, F), jnp.float32),
        pltpu.VMEM_SHARED((NPAD, F), jnp.float32),
        pltpu.SemaphoreType.DMA,
    ],
)
def _sc_scatter(ne, col, zsum, psum, idx_v, buf, acc, sem):
    cid = lax.axis_index("c")
    sid = lax.axis_index("s")
    nbase = pl.multiple_of(sid * NPT, NPT)

    def zinit(j, _):
        b = pl.multiple_of(nbase + j * GB, GB)
        pltpu.sync_copy(zsum.at[pl.ds(b, GB)], buf)
        pltpu.sync_copy(buf, acc.at[pl.ds(b, GB)])
        return 0

    lax.fori_loop(0, NPT // GB, zinit, 0)
    plsc.subcore_barrier()

    base_w = (cid * NS + sid) * EPW

    def body(i, _):
        base = pl.multiple_of(base_w + i * GB, GB)
        pltpu.sync_copy(col.at[pl.ds(base, GB)], idx_v)
        cp = pltpu.async_copy(ne.at[pl.ds(base, GB)], buf, sem)
        cp.wait()
        pltpu.sync_copy(buf, acc.at[idx_v], add=True)
        return 0

    lax.fori_loop(0, NCHUNK, body, 0)
    plsc.subcore_barrier()

    def wb(j, _):
        b = pl.multiple_of(nbase + j * GB, GB)
        pltpu.sync_copy(acc.at[pl.ds(b, GB)], buf)
        pltpu.sync_copy(buf, psum.at[cid, pl.ds(b, GB)])
        return 0

    lax.fori_loop(0, NPT // GB, wb, 0)


@functools.partial(
    pl.kernel,
    out_type=jax.ShapeDtypeStruct((NC, NPAD, F), jnp.float32),
    mesh=_mesh,
    scratch_types=[
        pltpu.VMEM((GB,), jnp.int32),
        pltpu.VMEM((GB, F), jnp.float32),
        pltpu.VMEM((GB, F), jnp.float32),
        pltpu.VMEM_SHARED((NPAD, F), jnp.float32),
        pltpu.SemaphoreType.DMA,
    ],
)
def _sc_count(col, zsum, ones, pcnt, idx_v, buf, ones_v, acc, sem):
    cid = lax.axis_index("c")
    sid = lax.axis_index("s")
    nbase = pl.multiple_of(sid * NPT, NPT)

    def zinit(j, _):
        b = pl.multiple_of(nbase + j * GB, GB)
        pltpu.sync_copy(zsum.at[pl.ds(b, GB)], buf)
        pltpu.sync_copy(buf, acc.at[pl.ds(b, GB)])
        return 0

    lax.fori_loop(0, NPT // GB, zinit, 0)
    pltpu.sync_copy(ones, ones_v)
    plsc.subcore_barrier()

    base_w = (cid * NS + sid) * EPW

    def body(i, _):
        base = pl.multiple_of(base_w + i * GB, GB)
        pltpu.sync_copy(col.at[pl.ds(base, GB)], idx_v)
        pltpu.sync_copy(ones_v, acc.at[idx_v], add=True)
        return 0

    lax.fori_loop(0, NCHUNK, body, 0)
    plsc.subcore_barrier()

    def wb(j, _):
        b = pl.multiple_of(nbase + j * GB, GB)
        pltpu.sync_copy(acc.at[pl.ds(b, GB)], buf)
        pltpu.sync_copy(buf, pcnt.at[cid, pl.ds(b, GB)])
        return 0

    lax.fori_loop(0, NPT // GB, wb, 0)


_PREC = lax.Precision.DEFAULT


def _dot(a, b):
    return jnp.dot(a, b, preferred_element_type=jnp.float32, precision=_PREC)


def _proj_body(x_ref, wa_ref, wb_ref, xr_ref, xc_ref):
    x = x_ref[...]
    xr_ref[...] = _dot(x, wa_ref[...])
    xc_ref[...] = _dot(x, wb_ref[...])


def _edge_body(gr_ref, gc_ref, ea_ref, w1_ref, b1_ref, w2_ref, b2_ref,
               w3_ref, b3_ref, w4_ref, b4_ref, ne_ref, eo_ref):
    ea = ea_ref[...]
    h = jnp.maximum(gr_ref[...] + gc_ref[...] + _dot(ea, w1_ref[...])
                    + b1_ref[...], 0.0)
    h = jnp.maximum(_dot(h, w2_ref[...]) + b2_ref[...], 0.0)
    h = jnp.maximum(_dot(h, w3_ref[...]) + b3_ref[...], 0.0)
    ne = _dot(h, w4_ref[...]) + b4_ref[...]
    ne_ref[...] = ne
    eo_ref[...] = ea + ne


def _node_body(x_ref, p0_ref, p1_ref, c0_ref, c1_ref, wa_ref, wb_ref, b1_ref,
               w2_ref, b2_ref, w3_ref, b3_ref, w4_ref, b4_ref, no_ref):
    x = x_ref[...]
    cnt = jnp.maximum(c0_ref[...][:, :1] + c1_ref[...][:, :1], 1.0)
    agg = (p0_ref[...] + p1_ref[...]) / cnt
    h = jnp.maximum(_dot(x, wa_ref[...]) + _dot(agg, wb_ref[...])
                    + b1_ref[...], 0.0)
    h = jnp.maximum(_dot(h, w2_ref[...]) + b2_ref[...], 0.0)
    h = jnp.maximum(_dot(h, w3_ref[...]) + b3_ref[...], 0.0)
    no_ref[...] = x + _dot(h, w4_ref[...]) + b4_ref[...]


def _wspec(shape):
    return pl.BlockSpec(shape, lambda i: (0,) * len(shape))


def kernel(x, edge_index, edge_attr, u, batch, We1, be1, We2, be2, We3, be3,
           We4, be4, Wn1, bn1, Wn2, bn2, Wn3, bn3, Wn4, bn4):
    row = edge_index[0]
    col = edge_index[1]

    BN = 1000
    xr, xc = pl.pallas_call(
        _proj_body,
        grid=(N // BN,),
        in_specs=[
            pl.BlockSpec((BN, F), lambda i: (i, 0)),
            _wspec((F, F)),
            _wspec((F, F)),
        ],
        out_specs=[
            pl.BlockSpec((BN, F), lambda i: (i, 0)),
            pl.BlockSpec((BN, F), lambda i: (i, 0)),
        ],
        out_shape=[
            jax.ShapeDtypeStruct((N, F), jnp.float32),
            jax.ShapeDtypeStruct((N, F), jnp.float32),
        ],
    )(x, We1[:F], We1[F:2 * F])

    gr, gc = _sc_gather(xr, xc, row, col)

    zsum = jnp.zeros((NPAD, F), jnp.float32)
    ones = jnp.ones((GB, F), jnp.float32)
    pcnt = _sc_count(col, zsum, ones)

    BE = 2000
    eb = pl.BlockSpec((BE, F), lambda i: (i, 0))
    bias = _wspec((1, F))
    ne, eout = pl.pallas_call(
        _edge_body,
        grid=(E // BE,),
        in_specs=[eb, eb, eb, _wspec((F, F)), bias, _wspec((F, F)), bias,
                  _wspec((F, F)), bias, _wspec((F, F)), bias],
        out_specs=[eb, eb],
        out_shape=[
            jax.ShapeDtypeStruct((E, F), jnp.float32),
            jax.ShapeDtypeStruct((E, F), jnp.float32),
        ],
    )(gr, gc, edge_attr, We1[2 * F:], be1[None], We2, be2[None], We3,
      be3[None], We4, be4[None])

    psum = _sc_scatter(ne, col, zsum)

    nb = pl.BlockSpec((BN, F), lambda i: (i, 0))
    nout = pl.pallas_call(
        _node_body,
        grid=(N // BN,),
        in_specs=[nb, nb, nb, nb, nb, _wspec((F, F)), _wspec((F, F)), bias,
                  _wspec((F, F)), bias, _wspec((F, F)), bias, _wspec((F, F)),
                  bias],
        out_specs=nb,
        out_shape=jax.ShapeDtypeStruct((N, F), jnp.float32),
    )(x, psum[0, :N], psum[1, :N], pcnt[0, :N], pcnt[1, :N], Wn1[:F],
      Wn1[F:], bn1[None], Wn2, bn2[None], Wn3, bn3[None], Wn4, bn4[None])

    return nout, eout


# speedup vs baseline: 3.6212x; 1.0977x over previous
import functools
import jax
import jax.numpy as jnp
from jax import lax
from jax.experimental import pallas as pl
from jax.experimental.pallas import tpu as pltpu, tpu_sc as plsc

N = 10000
E = 320000
F = 128
NC = 2
NS = 16
NW = NC * NS
GB = 80
CPW = E // (NW * GB)
CH0 = 63
CH1 = CPW - CH0
H0 = NW * CH0 * GB
H1 = NW * CH1 * GB
NPAD = 10240
NPT = NPAD // NS

_mesh = plsc.VectorSubcoreMesh(core_axis_name="c", subcore_axis_name="s")


def _make_gather(nch):
    eh = NW * nch * GB

    @functools.partial(
        pl.kernel,
        out_type=(
            jax.ShapeDtypeStruct((eh, F), jnp.float32),
            jax.ShapeDtypeStruct((eh, F), jnp.float32),
        ),
        mesh=_mesh,
        scratch_types=[
            pltpu.VMEM((GB,), jnp.int32),
            pltpu.VMEM((GB,), jnp.int32),
            pltpu.VMEM((GB,), jnp.int32),
            pltpu.VMEM((GB,), jnp.int32),
            pltpu.VMEM((GB, F), jnp.float32),
            pltpu.VMEM((GB, F), jnp.float32),
            pltpu.VMEM((GB, F), jnp.float32),
            pltpu.VMEM((GB, F), jnp.float32),
            pltpu.SemaphoreType.DMA,
            pltpu.SemaphoreType.DMA,
            pltpu.SemaphoreType.DMA,
            pltpu.SemaphoreType.DMA,
        ],
    )
    def gather(tab_r, tab_c, row, col, gr, gc, idx_r0, idx_c0, idx_r1,
               idx_c1, buf_r0, buf_c0, buf_r1, buf_c1, sem_r0, sem_c0,
               sem_r1, sem_c1):
        cid = lax.axis_index("c")
        sid = lax.axis_index("s")
        base_w = (cid * NS + sid) * (nch * GB)

        def chunk(base, idx_r, idx_c, buf_r, buf_c, sem_r, sem_c):
            pltpu.sync_copy(row.at[pl.ds(base, GB)], idx_r)
            pltpu.sync_copy(col.at[pl.ds(base, GB)], idx_c)
            cp_r = pltpu.async_copy(tab_r.at[idx_r], buf_r, sem_r)
            cp_c = pltpu.async_copy(tab_c.at[idx_c], buf_c, sem_c)
            return cp_r, cp_c

        def drain(base, buf_r, buf_c, cps):
            cps[0].wait()
            cps[1].wait()
            pltpu.sync_copy(buf_r, gr.at[pl.ds(base, GB)])
            pltpu.sync_copy(buf_c, gc.at[pl.ds(base, GB)])

        def body(i, _):
            base_a = pl.multiple_of(base_w + (2 * i) * GB, GB)
            base_b = pl.multiple_of(base_w + (2 * i + 1) * GB, GB)
            cps_a = chunk(base_a, idx_r0, idx_c0, buf_r0, buf_c0, sem_r0,
                          sem_c0)
            cps_b = chunk(base_b, idx_r1, idx_c1, buf_r1, buf_c1, sem_r1,
                          sem_c1)
            drain(base_a, buf_r0, buf_c0, cps_a)
            drain(base_b, buf_r1, buf_c1, cps_b)
            return 0

        lax.fori_loop(0, nch // 2, body, 0)
        if nch % 2:
            base_l = pl.multiple_of(base_w + (nch - 1) * GB, GB)
            cps_l = chunk(base_l, idx_r0, idx_c0, buf_r0, buf_c0, sem_r0,
                          sem_c0)
            drain(base_l, buf_r0, buf_c0, cps_l)

    return gather


_gather0 = _make_gather(CH0)
_gather1 = _make_gather(CH1)


def _make_scatter(nch):
    @functools.partial(
        pl.kernel,
        out_type=jax.ShapeDtypeStruct((NC, NPAD, F), jnp.float32),
        mesh=_mesh,
        scratch_types=[
            pltpu.VMEM((GB,), jnp.int32),
            pltpu.VMEM((GB,), jnp.int32),
            pltpu.VMEM((GB, F), jnp.float32),
            pltpu.VMEM((GB, F), jnp.float32),
            pltpu.VMEM_SHARED((NPAD, F), jnp.float32),
            pltpu.SemaphoreType.DMA,
            pltpu.SemaphoreType.DMA,
        ],
    )
    def scatter(ne, col, zsum, psum, idx0, idx1, buf0, buf1, acc, sem0,
                sem1):
        cid = lax.axis_index("c")
        sid = lax.axis_index("s")
        nbase = pl.multiple_of(sid * NPT, NPT)

        def zinit(j, _):
            b = pl.multiple_of(nbase + j * GB, GB)
            pltpu.sync_copy(zsum.at[pl.ds(b, GB)], buf0)
            pltpu.sync_copy(buf0, acc.at[pl.ds(b, GB)])
            return 0

        lax.fori_loop(0, NPT // GB, zinit, 0)
        plsc.subcore_barrier()

        base_w = (cid * NS + sid) * (nch * GB)

        def start(base, idx_v, bufx, semx):
            pltpu.sync_copy(col.at[pl.ds(base, GB)], idx_v)
            return pltpu.async_copy(ne.at[pl.ds(base, GB)], bufx, semx)

        def finish(cp, idx_v, bufx):
            cp.wait()
            pltpu.sync_copy(bufx, acc.at[idx_v], add=True)

        def body(i, _):
            base_a = pl.multiple_of(base_w + (2 * i) * GB, GB)
            base_b = pl.multiple_of(base_w + (2 * i + 1) * GB, GB)
            cp_a = start(base_a, idx0, buf0, sem0)
            cp_b = start(base_b, idx1, buf1, sem1)
            finish(cp_a, idx0, buf0)
            finish(cp_b, idx1, buf1)
            return 0

        lax.fori_loop(0, nch // 2, body, 0)
        if nch % 2:
            base_l = pl.multiple_of(base_w + (nch - 1) * GB, GB)
            finish(start(base_l, idx0, buf0, sem0), idx0, buf0)
        plsc.subcore_barrier()

        def wb(j, _):
            b = pl.multiple_of(nbase + j * GB, GB)
            pltpu.sync_copy(acc.at[pl.ds(b, GB)], buf0)
            pltpu.sync_copy(buf0, psum.at[cid, pl.ds(b, GB)])
            return 0

        lax.fori_loop(0, NPT // GB, wb, 0)

    return scatter


_scatter0 = _make_scatter(CH0)
_scatter1 = _make_scatter(CH1)


@functools.partial(
    pl.kernel,
    out_type=jax.ShapeDtypeStruct((NC, NPAD, F), jnp.float32),
    mesh=_mesh,
    scratch_types=[
        pltpu.VMEM((GB,), jnp.int32),
        pltpu.VMEM((GB,), jnp.int32),
        pltpu.VMEM((GB, F), jnp.float32),
        pltpu.VMEM((GB, F), jnp.float32),
        pltpu.VMEM_SHARED((NPAD, F), jnp.float32),
        pltpu.SemaphoreType.DMA,
        pltpu.SemaphoreType.DMA,
    ],
)
def _sc_count(col, zsum, ones, pcnt, idx0, idx1, buf, ones_v, acc, sem0,
              sem1):
    cid = lax.axis_index("c")
    sid = lax.axis_index("s")
    nbase = pl.multiple_of(sid * NPT, NPT)

    def zinit(j, _):
        b = pl.multiple_of(nbase + j * GB, GB)
        pltpu.sync_copy(zsum.at[pl.ds(b, GB)], buf)
        pltpu.sync_copy(buf, acc.at[pl.ds(b, GB)])
        return 0

    lax.fori_loop(0, NPT // GB, zinit, 0)
    pltpu.sync_copy(ones, ones_v)
    plsc.subcore_barrier()

    base_w = (cid * NS + sid) * (CPW * GB)

    def body(i, _):
        base_a = pl.multiple_of(base_w + (2 * i) * GB, GB)
        base_b = pl.multiple_of(base_w + (2 * i + 1) * GB, GB)
        cp_a = pltpu.async_copy(col.at[pl.ds(base_a, GB)], idx0, sem0)
        cp_b = pltpu.async_copy(col.at[pl.ds(base_b, GB)], idx1, sem1)
        cp_a.wait()
        pltpu.sync_copy(ones_v, acc.at[idx0], add=True)
        cp_b.wait()
        pltpu.sync_copy(ones_v, acc.at[idx1], add=True)
        return 0

    lax.fori_loop(0, CPW // 2, body, 0)
    base_l = pl.multiple_of(base_w + (CPW - 1) * GB, GB)
    pltpu.sync_copy(col.at[pl.ds(base_l, GB)], idx0)
    pltpu.sync_copy(ones_v, acc.at[idx0], add=True)
    plsc.subcore_barrier()

    def wb(j, _):
        b = pl.multiple_of(nbase + j * GB, GB)
        pltpu.sync_copy(acc.at[pl.ds(b, GB)], buf)
        pltpu.sync_copy(buf, pcnt.at[cid, pl.ds(b, GB)])
        return 0

    lax.fori_loop(0, NPT // GB, wb, 0)


def _dot(a, b):
    return jnp.dot(a, b, preferred_element_type=jnp.float32)


def _proj_body(x_ref, wa_ref, wb_ref, xr_ref, xc_ref):
    x = x_ref[...]
    xr_ref[...] = _dot(x, wa_ref[...])
    xc_ref[...] = _dot(x, wb_ref[...])


def _edge_body(gr_ref, gc_ref, ea_ref, w1_ref, b1_ref, w2_ref, b2_ref,
               w3_ref, b3_ref, w4_ref, b4_ref, ne_ref, eo_ref):
    ea = ea_ref[...]
    h = jnp.maximum(gr_ref[...] + gc_ref[...] + _dot(ea, w1_ref[...])
                    + b1_ref[...], 0.0)
    h = jnp.maximum(_dot(h, w2_ref[...]) + b2_ref[...], 0.0)
    h = jnp.maximum(_dot(h, w3_ref[...]) + b3_ref[...], 0.0)
    ne = _dot(h, w4_ref[...]) + b4_ref[...]
    ne_ref[...] = ne
    eo_ref[...] = ea + ne


def _node_body(x_ref, p00_ref, p01_ref, p10_ref, p11_ref, c0_ref, c1_ref,
               wa_ref, wb_ref, b1_ref, w2_ref, b2_ref, w3_ref, b3_ref,
               w4_ref, b4_ref, no_ref):
    x = x_ref[...]
    cnt = jnp.maximum(c0_ref[...][:, :1] + c1_ref[...][:, :1], 1.0)
    agg = (p00_ref[...] + p01_ref[...] + p10_ref[...] + p11_ref[...]) / cnt
    h = jnp.maximum(_dot(x, wa_ref[...]) + _dot(agg, wb_ref[...])
                    + b1_ref[...], 0.0)
    h = jnp.maximum(_dot(h, w2_ref[...]) + b2_ref[...], 0.0)
    h = jnp.maximum(_dot(h, w3_ref[...]) + b3_ref[...], 0.0)
    no_ref[...] = x + _dot(h, w4_ref[...]) + b4_ref[...]


def _wspec(shape):
    return pl.BlockSpec(shape, lambda i: (0,) * len(shape))


def _edge_mlp(gr, gc, ea, We1c, be1, We2, be2, We3, be3, We4, be4):
    eh = gr.shape[0]
    be = NW * GB
    eb = pl.BlockSpec((be, F), lambda i: (i, 0))
    bias = _wspec((1, F))
    return pl.pallas_call(
        _edge_body,
        grid=(eh // be,),
        in_specs=[eb, eb, eb, _wspec((F, F)), bias, _wspec((F, F)), bias,
                  _wspec((F, F)), bias, _wspec((F, F)), bias],
        out_specs=[eb, eb],
        out_shape=[
            jax.ShapeDtypeStruct((eh, F), jnp.float32),
            jax.ShapeDtypeStruct((eh, F), jnp.float32),
        ],
    )(gr, gc, ea, We1c, be1[None], We2, be2[None], We3, be3[None], We4,
      be4[None])


def kernel(x, edge_index, edge_attr, u, batch, We1, be1, We2, be2, We3, be3,
           We4, be4, Wn1, bn1, Wn2, bn2, Wn3, bn3, Wn4, bn4):
    row = edge_index[0]
    col = edge_index[1]
    row0, row1 = row[:H0], row[H0:]
    col0, col1 = col[:H0], col[H0:]
    ea0, ea1 = edge_attr[:H0], edge_attr[H0:]

    BN = 1000
    xr, xc = pl.pallas_call(
        _proj_body,
        grid=(N // BN,),
        in_specs=[
            pl.BlockSpec((BN, F), lambda i: (i, 0)),
            _wspec((F, F)),
            _wspec((F, F)),
        ],
        out_specs=[
            pl.BlockSpec((BN, F), lambda i: (i, 0)),
            pl.BlockSpec((BN, F), lambda i: (i, 0)),
        ],
        out_shape=[
            jax.ShapeDtypeStruct((N, F), jnp.float32),
            jax.ShapeDtypeStruct((N, F), jnp.float32),
        ],
    )(x, We1[:F], We1[F:2 * F])

    zsum = jnp.zeros((NPAD, F), jnp.float32)
    ones = jnp.ones((GB, F), jnp.float32)
    pcnt = _sc_count(col, zsum, ones)

    g0r, g0c = _gather0(xr, xc, row0, col0)
    ne0, eo0 = _edge_mlp(g0r, g0c, ea0, We1[2 * F:], be1, We2, be2, We3,
                         be3, We4, be4)
    g1r, g1c = _gather1(xr, xc, row1, col1)
    ps0 = _scatter0(ne0, col0, zsum)
    ne1, eo1 = _edge_mlp(g1r, g1c, ea1, We1[2 * F:], be1, We2, be2, We3,
                         be3, We4, be4)
    ps1 = _scatter1(ne1, col1, zsum)

    nb = pl.BlockSpec((BN, F), lambda i: (i, 0))
    bias = _wspec((1, F))
    nout = pl.pallas_call(
        _node_body,
        grid=(N // BN,),
        in_specs=[nb, nb, nb, nb, nb, nb, nb, _wspec((F, F)),
                  _wspec((F, F)), bias, _wspec((F, F)), bias, _wspec((F, F)),
                  bias, _wspec((F, F)), bias],
        out_specs=nb,
        out_shape=jax.ShapeDtypeStruct((N, F), jnp.float32),
    )(x, ps0[0, :N], ps0[1, :N], ps1[0, :N], ps1[1, :N], pcnt[0, :N],
      pcnt[1, :N], Wn1[:F], Wn1[F:], bn1[None], Wn2, bn2[None], Wn3,
      bn3[None], Wn4, bn4[None])

    eout = jnp.concatenate([eo0, eo1], axis=0)
    return nout, eout
